# baseline (speedup 1.0000x reference)
import functools

import jax
import jax.numpy as jnp
from jax.experimental import pallas as pl
from jax.experimental.pallas import tpu as pltpu

B, D, K, P = 8, 64, 512, 256
BP = B * P

_DEF = jax.lax.Precision.DEFAULT
_DN0 = (((0,), (0,)), ((), ()))
_DN1 = (((1,), (0,)), ((), ()))


def _split3(m):
    hi = jnp.asarray(m, jnp.bfloat16)
    r = m - hi.astype(jnp.float32)
    mid = jnp.asarray(r, jnp.bfloat16)
    lo = jnp.asarray(r - mid.astype(jnp.float32), jnp.bfloat16)
    return hi, mid, lo


def _vq_kernel(x_ref, e_ref, q_ref, enc_ref, loss_ref, perp_ref, xx_ref):
    b = pl.program_id(0)
    xx_ref[:, pl.ds(b * P, P)] = x_ref[0].reshape(D, P)

    @pl.when(b == B - 1)
    def _compute():
        e = e_ref[...]
        xx = xx_ref[...]

        e_hi, e_mid, e_lo = _split3(e)
        x_hi = jnp.asarray(xx, jnp.bfloat16)
        x_lo = jnp.asarray(xx - x_hi.astype(jnp.float32), jnp.bfloat16)

        def _dot0(l, r):
            return jax.lax.dot_general(l, r, _DN0, precision=_DEF,
                                       preferred_element_type=jnp.float32)

        def _dot1(l, r):
            return jax.lax.dot_general(l, r, _DN1, precision=_DEF,
                                       preferred_element_type=jnp.float32)

        s = _dot0(e_hi, x_hi) + (_dot0(e_hi, x_lo) + _dot0(e_mid, x_hi))
        e2 = _dot0(e * e, jnp.ones((D, 1), jnp.float32))
        a = e2 - 2.0 * s

        iota_k = jax.lax.broadcasted_iota(jnp.int32, (K, BP), 0)
        inf = jnp.float32(jnp.inf)
        i0 = jnp.argmin(a, axis=0, keepdims=True)
        a1m = jnp.where(iota_k == i0, inf, a)
        i1 = jnp.argmin(a1m, axis=0, keepdims=True)

        oh0 = (iota_k == i0).astype(jnp.bfloat16)
        oh1 = (iota_k == i1).astype(jnp.bfloat16)
        oh = jnp.concatenate([oh0, oh1], axis=1)

        qq = _dot1(e_hi, oh) + (_dot1(e_mid, oh) + _dot1(e_lo, oh))

        x2 = jnp.concatenate([xx, xx], axis=1)
        acc = jnp.zeros((1, 2 * BP), jnp.float32)
        for d in range(D):
            dd = qq[d:d + 1, :] - x2[d:d + 1, :]
            acc = acc + dd * dd
        sq = jnp.sqrt(acc)
        s0 = sq[:, :BP]
        s1 = sq[:, BP:]

        w1 = (s1 < s0) | ((s1 == s0) & (i1 < i0))
        enc = jnp.where(w1, oh[:, BP:], oh[:, :BP]).astype(jnp.float32)
        qw = jnp.where(w1, qq[:, BP:], qq[:, :BP])
        st = xx + (qw - xx)
        for bb in range(B):
            sl = slice(bb * P, (bb + 1) * P)
            enc_ref[bb] = enc[:, sl]
            q_ref[bb] = st[:, sl]

        loss_sum = jnp.sum(jnp.where(w1, acc[:, BP:], acc[:, :BP]))
        loss_ref[...] = jnp.full((1, 1), loss_sum / (B * D * P), jnp.float32)
        counts = jnp.sum(enc, axis=1, keepdims=True)
        avg = counts / (B * P)
        ent = jnp.sum(avg * jnp.log(avg + 1e-10))
        perp_ref[...] = jnp.full((1, 1), jnp.exp(-ent) / K, jnp.float32)


@functools.partial(jax.jit, static_argnames=())
def _vq_call(x, e):
    return pl.pallas_call(
        _vq_kernel,
        grid=(B,),
        in_specs=[
            pl.BlockSpec((1, D, 16, 16), lambda b: (b, 0, 0, 0)),
            pl.BlockSpec((D, K), lambda b: (0, 0)),
        ],
        out_specs=[
            pl.BlockSpec((B, D, P), lambda b: (0, 0, 0)),
            pl.BlockSpec((B, K, P), lambda b: (0, 0, 0)),
            pl.BlockSpec((1, 1), lambda b: (0, 0)),
            pl.BlockSpec((1, 1), lambda b: (0, 0)),
        ],
        out_shape=[
            jax.ShapeDtypeStruct((B, D, P), jnp.float32),
            jax.ShapeDtypeStruct((B, K, P), jnp.float32),
            jax.ShapeDtypeStruct((1, 1), jnp.float32),
            jax.ShapeDtypeStruct((1, 1), jnp.float32),
        ],
        scratch_shapes=[
            pltpu.VMEM((D, BP), jnp.float32),
        ],
    )(x, e)


def kernel(input, embedding):
    b, d, h, w = input.shape
    e = embedding[:, :, 0]
    q, enc, loss, perp = _vq_call(input, e)
    return (q.reshape(b, d, h, w),
            enc.reshape(b, K, h, w),
            loss.reshape(()),
            perp.reshape(1))


# speedup vs baseline: 1.2861x; 1.2861x over previous
import functools

import jax
import jax.numpy as jnp
from jax.experimental import pallas as pl

B, D, K, P = 8, 64, 512, 256
BP = B * P

_DEF = jax.lax.Precision.DEFAULT


def _split3(m):
    hi = jnp.asarray(m, jnp.bfloat16)
    r = m - hi.astype(jnp.float32)
    mid = jnp.asarray(r, jnp.bfloat16)
    lo = jnp.asarray(r - mid.astype(jnp.float32), jnp.bfloat16)
    return hi, mid, lo


def _vq_kernel(x_ref, e_ref, q_ref, enc_ref, loss_ref, perp_ref):
    e = e_ref[...]
    x3 = x_ref[...]
    xx = jnp.concatenate([x3[bb] for bb in range(B)], axis=1)

    e_hi, e_mid, e_lo = _split3(e)
    x_hi = jnp.asarray(xx, jnp.bfloat16)
    x_lo = jnp.asarray(xx - x_hi.astype(jnp.float32), jnp.bfloat16)
    dn = (((0,), (0,)), ((), ()))

    def _bdot(l, r):
        return jax.lax.dot_general(l, r, dn, precision=_DEF,
                                   preferred_element_type=jnp.float32)

    s = _bdot(e_hi, x_hi) + (_bdot(e_hi, x_lo) + _bdot(e_mid, x_hi))
    e2 = _bdot(e * e, jnp.ones((D, 1), jnp.float32))
    a = e2 - 2.0 * s

    iota_k = jax.lax.broadcasted_iota(jnp.int32, (K, BP), 0)
    inf = jnp.float32(jnp.inf)
    i0 = jnp.argmin(a, axis=0, keepdims=True)
    a1m = jnp.where(iota_k == i0, inf, a)
    i1 = jnp.argmin(a1m, axis=0, keepdims=True)

    oh0 = (iota_k == i0).astype(jnp.bfloat16)
    oh1 = (iota_k == i1).astype(jnp.bfloat16)
    oh = jnp.concatenate([oh0, oh1], axis=1)

    dg = (((1,), (0,)), ((), ()))

    def _gdot(l, r):
        return jax.lax.dot_general(l, r, dg, precision=_DEF,
                                   preferred_element_type=jnp.float32)

    qq = _gdot(e_hi, oh) + (_gdot(e_mid, oh) + _gdot(e_lo, oh))

    x2 = jnp.concatenate([xx, xx], axis=1)
    acc = jnp.zeros((1, 2 * BP), jnp.float32)
    for d in range(D):
        dd = qq[d:d + 1, :] - x2[d:d + 1, :]
        acc = acc + dd * dd
    sq = jnp.sqrt(acc)
    s0 = sq[:, :BP]
    s1 = sq[:, BP:]

    w1 = (s1 < s0) | ((s1 == s0) & (i1 < i0))
    enc = jnp.where(w1, oh[:, BP:], oh[:, :BP]).astype(jnp.float32)
    qw = jnp.where(w1, qq[:, BP:], qq[:, :BP])
    st = xx + (qw - xx)
    for bb in range(B):
        sl = slice(bb * P, (bb + 1) * P)
        enc_ref[bb] = enc[:, sl]
        q_ref[bb] = st[:, sl]

    loss_sum = jnp.sum(jnp.where(w1, acc[:, BP:], acc[:, :BP]))
    loss_ref[...] = jnp.full((1, 1), loss_sum / (B * D * P), jnp.float32)
    counts = jnp.sum(enc, axis=1, keepdims=True)
    avg = counts / (B * P)
    ent = jnp.sum(avg * jnp.log(avg + 1e-10))
    perp_ref[...] = jnp.full((1, 1), jnp.exp(-ent) / K, jnp.float32)


@functools.partial(jax.jit, static_argnames=())
def _vq_call(x, e):
    return pl.pallas_call(
        _vq_kernel,
        out_shape=[
            jax.ShapeDtypeStruct((B, D, P), jnp.float32),
            jax.ShapeDtypeStruct((B, K, P), jnp.float32),
            jax.ShapeDtypeStruct((1, 1), jnp.float32),
            jax.ShapeDtypeStruct((1, 1), jnp.float32),
        ],
    )(x, e)


def kernel(input, embedding):
    b, d, h, w = input.shape
    x = input.reshape(b, d, h * w)
    e = embedding[:, :, 0]
    q, enc, loss, perp = _vq_call(x, e)
    return (q.reshape(b, d, h, w),
            enc.reshape(b, K, h, w),
            loss.reshape(()),
            perp.reshape(1))
